# Initial kernel scaffold; baseline (speedup 1.0000x reference)
import jax
import jax.numpy as jnp
from jax.experimental import pallas as pl


def kernel(x, emb):
    raise NotImplementedError("write your pallas kernel here")


# speedup vs baseline: 10.1653x; 10.1653x over previous
import jax
import jax.numpy as jnp
from jax import lax
from jax.experimental import pallas as pl
from jax.experimental.pallas import tpu as pltpu
from jax.experimental.pallas import tpu_sc as plsc

B, C, T, D, V = 16, 8, 4096, 64, 2048
K = 128
NC, NS = 2, 16
NW = NC * NS
TOK_PER_W = (B * T) // NW
CHUNKS = TOK_PER_W // K


def _embed_body(x_hbm, emb_hbm, out_hbm, idx_v, rows_v, out_v, sem):
    cid = lax.axis_index("core")
    sid = lax.axis_index("sub")
    wid = sid * NC + cid
    b = wid // 2
    t_half = (wid % 2) * TOK_PER_W

    def chunk_body(i, carry):
        t0 = t_half + i * K
        pltpu.sync_copy(x_hbm.at[b, :, pl.ds(t0, K)], idx_v)
        for c in range(1, C):
            for j in range(K // 16):
                sl = pl.ds(j * 16, 16)
                idx_v[c, sl] = idx_v[c, sl] + (c * V)
        copies = [
            pltpu.async_copy(emb_hbm.at[idx_v.at[c]], rows_v.at[c], sem)
            for c in range(C)
        ]
        for cp in copies:
            cp.wait()
        def tok_body(k, carry2):
            for dd in range(D // 16):
                sl = pl.ds(dd * 16, 16)
                acc = rows_v[0, k, sl]
                for c in range(1, C):
                    acc = acc + rows_v[c, k, sl]
                out_v[k, sl] = acc * (1.0 / C)
            return carry2
        lax.fori_loop(0, K, tok_body, 0, unroll=2)
        pltpu.sync_copy(out_v, out_hbm.at[b, pl.ds(t0, K)])
        return carry

    lax.fori_loop(0, CHUNKS, chunk_body, 0)


_mesh = plsc.VectorSubcoreMesh(
    core_axis_name="core", subcore_axis_name="sub",
    num_cores=NC, num_subcores=NS)

_embed = pl.kernel(
    _embed_body,
    out_type=jax.ShapeDtypeStruct((B, T, D), jnp.float32),
    mesh=_mesh,
    scratch_types=[
        pltpu.VMEM((C, K), jnp.int32),
        pltpu.VMEM((C, K, D), jnp.float32),
        pltpu.VMEM((K, D), jnp.float32),
        pltpu.SemaphoreType.DMA,
    ],
    compiler_params=pltpu.CompilerParams(use_tc_tiling_on_sc=False),
)


def kernel(x, emb):
    return _embed(x.astype(jnp.int32), emb)


# speedup vs baseline: 13.8510x; 1.3626x over previous
import jax
import jax.numpy as jnp
from jax import lax
from jax.experimental import pallas as pl
from jax.experimental.pallas import tpu as pltpu
from jax.experimental.pallas import tpu_sc as plsc

B, C, T, D, V = 16, 8, 4096, 64, 2048
K = 64
NC, NS = 2, 16
NW = NC * NS
TOK_PER_W = (B * T) // NW
CHUNKS = TOK_PER_W // K


def _embed_body(x_hbm, emb_hbm, out_hbm, idx_v, rows0, rows1, outv0, outv1,
                sg0, sg1, so0, so1):
    cid = lax.axis_index("core")
    sid = lax.axis_index("sub")
    wid = sid * NC + cid
    b = wid // 2
    t_half = (wid % 2) * TOK_PER_W
    rows = (rows0, rows1)
    outv = (outv0, outv1)
    sg = (sg0, sg1)
    so = (so0, so1)

    for c in range(C):
        pltpu.async_copy(
            x_hbm.at[pl.ds(b * (C * T) + c * T + t_half, TOK_PER_W)],
            idx_v.at[c], sg0)
    for c in range(C):
        pltpu.make_async_copy(
            x_hbm.at[pl.ds(b * (C * T) + c * T + t_half, TOK_PER_W)],
            idx_v.at[c], sg0).wait()

    def off_body(j, carry):
        sl = pl.ds(j * 16, 16)
        for c in range(1, C):
            idx_v[c, sl] = idx_v[c, sl] + (c * V)
        return carry
    lax.fori_loop(0, TOK_PER_W // 16, off_body, 0, unroll=2)

    def fire_gathers(i, p):
        loc = i * K
        for c in range(C):
            pltpu.async_copy(
                emb_hbm.at[idx_v.at[c, pl.ds(loc, K)]], rows[p].at[c], sg[p])

    def drain_gathers(i, p):
        loc = i * K
        for c in range(C):
            pltpu.make_async_copy(
                emb_hbm.at[idx_v.at[c, pl.ds(loc, K)]], rows[p].at[c],
                sg[p]).wait()

    def accum(p):
        def tok_body(k, carry):
            for dd in range(D // 16):
                sl = pl.ds(dd * 16, 16)
                acc = rows[p][0, k, sl]
                for c in range(1, C):
                    acc = acc + rows[p][c, k, sl]
                outv[p][k, sl] = acc * (1.0 / C)
            return carry
        lax.fori_loop(0, K, tok_body, 0, unroll=2)

    def fire_out(i, p):
        pltpu.async_copy(outv[p], out_hbm.at[b, pl.ds(t_half + i * K, K)],
                         so[p])

    def wait_out(i, p):
        pltpu.make_async_copy(outv[p], out_hbm.at[b, pl.ds(t_half + i * K, K)],
                              so[p]).wait()

    fire_gathers(0, 0)

    def pair_body(i, carry):
        ii = 2 * i
        fire_gathers(ii + 1, 1)
        drain_gathers(ii, 0)
        @pl.when(i > 0)
        def _():
            wait_out(ii - 2, 0)
        accum(0)
        fire_out(ii, 0)
        @pl.when(ii + 2 < CHUNKS)
        def _():
            fire_gathers(ii + 2, 0)
        drain_gathers(ii + 1, 1)
        @pl.when(i > 0)
        def _():
            wait_out(ii - 1, 1)
        accum(1)
        fire_out(ii + 1, 1)
        return carry

    lax.fori_loop(0, CHUNKS // 2, pair_body, 0)
    wait_out(CHUNKS - 2, 0)
    wait_out(CHUNKS - 1, 1)


_mesh = plsc.VectorSubcoreMesh(
    core_axis_name="core", subcore_axis_name="sub",
    num_cores=NC, num_subcores=NS)

_embed = pl.kernel(
    _embed_body,
    out_type=jax.ShapeDtypeStruct((B, T, D), jnp.float32),
    mesh=_mesh,
    scratch_types=[
        pltpu.VMEM((C, TOK_PER_W), jnp.int32),
        pltpu.VMEM((C, K, D), jnp.float32),
        pltpu.VMEM((C, K, D), jnp.float32),
        pltpu.VMEM((K, D), jnp.float32),
        pltpu.VMEM((K, D), jnp.float32),
        pltpu.SemaphoreType.DMA,
        pltpu.SemaphoreType.DMA,
        pltpu.SemaphoreType.DMA,
        pltpu.SemaphoreType.DMA,
    ],
    compiler_params=pltpu.CompilerParams(use_tc_tiling_on_sc=False),
)


def kernel(x, emb):
    return _embed(x.reshape(B * C * T).astype(jnp.int32), emb)
